# baseline (speedup 1.0000x reference)
import functools

import jax
import jax.numpy as jnp
from jax import lax
from jax.experimental import pallas as pl
from jax.experimental.pallas import tpu as pltpu
from jax.experimental.pallas import tpu_sc as plsc

E = 16
N = 16384 * 2
NSUB = 16
CH = N // NSUB
SEG = CH // 16
NV = NSUB * 16
ROWS = CH // 128

_mesh = plsc.VectorSubcoreMesh(core_axis_name="c", subcore_axis_name="s")
_params = pltpu.CompilerParams(needs_layout_passes=False)


@functools.partial(
    pl.kernel,
    mesh=_mesh,
    compiler_params=_params,
    out_type=(
        jax.ShapeDtypeStruct((N,), jnp.float32),
        jax.ShapeDtypeStruct((N,), jnp.int32),
        jax.ShapeDtypeStruct((E,), jnp.float32),
    ),
    scratch_types=[
        pltpu.VMEM((CH // 128, 128), jnp.int32),
        pltpu.VMEM((CH // 128, 128), jnp.float32),
        pltpu.VMEM((16, E), jnp.int32),
        pltpu.VMEM((E,), jnp.int32),
        pltpu.VMEM((NSUB, E), jnp.int32),
        pltpu.VMEM((16, E), jnp.int32),
        pltpu.VMEM((16, E), jnp.int32),
        pltpu.VMEM((ROWS, 128), jnp.int32),
        pltpu.VMEM((ROWS, 128), jnp.int32),
        pltpu.VMEM((ROWS, 128), jnp.float32),
        pltpu.VMEM((E,), jnp.float32),
        pltpu.VMEM((CH,), jnp.int32),
        pltpu.VMEM((CH,), jnp.float32),
        pltpu.VMEM_SHARED((NSUB, E), jnp.int32),
        pltpu.VMEM_SHARED((N,), jnp.int32),
        pltpu.VMEM_SHARED((N,), jnp.float32),
        pltpu.SemaphoreType.DMA,
    ],
)
def _reorder_kernel(eids_hbm, scores_hbm, sc_out, idx_out, cnt_out,
                    ev, sv, h2d, bsv, blkv, g2d, d2d, loc_idx, loc_dst, loc_sc,
                    cnt_v, bi, bs, sh_blk, sh_idx, sh_sc, sem):
    c = lax.axis_index("c")
    s = lax.axis_index("s")
    base_elem = s * CH
    base_row = s * (CH // 128)
    pltpu.sync_copy(eids_hbm.at[pl.ds(base_row, CH // 128), :], ev)
    pltpu.sync_copy(scores_hbm.at[pl.ds(base_row, CH // 128), :], sv)

    iota = lax.iota(jnp.int32, 16)
    zeros = jnp.zeros((16,), jnp.int32)
    ones = jnp.ones((16,), jnp.int32)
    for r in range(16):
        h2d[r, :] = zeros
    seg = iota * SEG

    def hbody(t, cc):
        for u in range(2):
            idxs = seg + (t * 2 + u)
            v = plsc.load_gather(ev, [idxs >> 7, idxs & 127])
            plsc.addupdate_scatter(h2d, [iota, v], ones)
        return cc

    lax.fori_loop(0, SEG // 2, hbody, 0)
    bsum = jnp.zeros((16,), jnp.int32)
    for r in range(16):
        bsum = bsum + h2d[r, :]
    bsv[...] = bsum
    pltpu.sync_copy(bsv, sh_blk.at[s])
    plsc.subcore_barrier()

    pltpu.sync_copy(sh_blk, blkv)
    col = jnp.zeros((16,), jnp.int32)
    pre = jnp.zeros((16,), jnp.int32)
    for r in range(NSUB):
        row = blkv[r, :]
        col = col + row
        pre = pre + jnp.where(r < s, row, 0)
    base_e = plsc.cumsum(col) - col
    rg = base_e + pre
    rl = plsc.cumsum(bsum) - bsum
    half = N // 2
    lowcnt = jnp.minimum(half, rg + bsum) - jnp.minimum(half, rg)
    cut = jnp.sum(lowcnt)
    for lane in range(16):
        g2d[lane, :] = rg
        d2d[lane, :] = rg - rl
        hrow = h2d[lane, :]
        rg = rg + hrow
        rl = rl + hrow

    @pl.when((c == 0) & (s == 0))
    def _():
        cnt_v[...] = col.astype(jnp.float32)
        pltpu.sync_copy(cnt_v, cnt_out)

    def body(t, cc):
        for u in range(2):
            idxs = seg + (t * 2 + u)
            v = plsc.load_gather(ev, [idxs >> 7, idxs & 127])
            gdst = plsc.load_gather(g2d, [iota, v])
            dl = plsc.load_gather(d2d, [iota, v])
            ldst = gdst - dl
            sc = plsc.load_gather(sv, [idxs >> 7, idxs & 127])
            i0 = ldst >> 7
            i1 = ldst & 127
            plsc.store_scatter(loc_idx, [i0, i1], base_elem + idxs)
            plsc.store_scatter(loc_dst, [i0, i1], gdst)
            plsc.store_scatter(loc_sc, [i0, i1], sc)
            plsc.addupdate_scatter(g2d, [iota, v], ones)
        return cc

    lax.fori_loop(0, SEG // 2, body, 0)

    for j in range(ROWS):
        take = jnp.where(c == 0, j * 128 < cut, j * 128 + 128 > cut)

        @pl.when(take)
        def _(j=j):
            cp1 = pltpu.async_copy(loc_idx.at[j], sh_idx.at[loc_dst.at[j]], sem)
            cp2 = pltpu.async_copy(loc_sc.at[j], sh_sc.at[loc_dst.at[j]], sem)
            cp1.wait()
            cp2.wait()
    plsc.subcore_barrier()

    @pl.when((s >> 3) == c)
    def _():
        pltpu.sync_copy(sh_idx.at[pl.ds(base_elem, CH)], bi)
        pltpu.sync_copy(bi, idx_out.at[pl.ds(base_elem, CH)])
        pltpu.sync_copy(sh_sc.at[pl.ds(base_elem, CH)], bs)
        pltpu.sync_copy(bs, sc_out.at[pl.ds(base_elem, CH)])


@jax.jit
def _token_reorder(top_scores, selected_experts_indices):
    eids = selected_experts_indices.reshape(N // 128, 128)
    scores = top_scores.reshape(N // 128, 128)
    return _reorder_kernel(eids, scores)


def kernel(top_scores, selected_experts_indices):
    return _token_reorder(top_scores, selected_experts_indices)


# speedup vs baseline: 1.0044x; 1.0044x over previous
import functools

import jax
import jax.numpy as jnp
from jax import lax
from jax.experimental import pallas as pl
from jax.experimental.pallas import tpu as pltpu
from jax.experimental.pallas import tpu_sc as plsc

E = 16
N = 16384 * 2
NSUB = 16
CH = N // NSUB
SEG = CH // 16
NV = NSUB * 16
ROWS = CH // 128

_mesh = plsc.VectorSubcoreMesh(core_axis_name="c", subcore_axis_name="s")
_params = pltpu.CompilerParams(needs_layout_passes=False)


@functools.partial(
    pl.kernel,
    mesh=_mesh,
    compiler_params=_params,
    out_type=(
        jax.ShapeDtypeStruct((N,), jnp.float32),
        jax.ShapeDtypeStruct((N,), jnp.int32),
        jax.ShapeDtypeStruct((E,), jnp.float32),
    ),
    scratch_types=[
        pltpu.VMEM((CH // 128, 128), jnp.int32),
        pltpu.VMEM((CH // 128, 128), jnp.float32),
        pltpu.VMEM((16, E), jnp.int32),
        pltpu.VMEM((E,), jnp.int32),
        pltpu.VMEM((NSUB, E), jnp.int32),
        pltpu.VMEM((16, E), jnp.int32),
        pltpu.VMEM((16, E), jnp.int32),
        pltpu.VMEM((ROWS, 128), jnp.int32),
        pltpu.VMEM((ROWS, 128), jnp.int32),
        pltpu.VMEM((ROWS, 128), jnp.float32),
        pltpu.VMEM((E,), jnp.float32),
        pltpu.VMEM((CH,), jnp.int32),
        pltpu.VMEM((CH,), jnp.float32),
        pltpu.VMEM_SHARED((NSUB, E), jnp.int32),
        pltpu.VMEM_SHARED((N,), jnp.int32),
        pltpu.VMEM_SHARED((N,), jnp.float32),
        pltpu.SemaphoreType.DMA,
    ],
)
def _reorder_kernel(eids_hbm, scores_hbm, sc_out, idx_out, cnt_out,
                    ev, sv, h2d, bsv, blkv, g2d, d2d, loc_idx, loc_dst, loc_sc,
                    cnt_v, bi, bs, sh_blk, sh_idx, sh_sc, sem):
    c = lax.axis_index("c")
    s = lax.axis_index("s")
    base_elem = s * CH
    base_row = s * (CH // 128)
    pltpu.sync_copy(eids_hbm.at[pl.ds(base_row, CH // 128), :], ev)
    pltpu.sync_copy(scores_hbm.at[pl.ds(base_row, CH // 128), :], sv)

    iota = lax.iota(jnp.int32, 16)
    zeros = jnp.zeros((16,), jnp.int32)
    ones = jnp.ones((16,), jnp.int32)
    for r in range(16):
        h2d[r, :] = zeros
    seg = iota * SEG

    def hbody(t, cc):
        for u in range(2):
            idxs = seg + (t * 2 + u)
            v = plsc.load_gather(ev, [idxs >> 7, idxs & 127])
            plsc.addupdate_scatter(h2d, [iota, v], ones)
        return cc

    lax.fori_loop(0, SEG // 2, hbody, 0)
    bsum = jnp.zeros((16,), jnp.int32)
    for r in range(16):
        bsum = bsum + h2d[r, :]
    bsv[...] = bsum
    pltpu.sync_copy(bsv, sh_blk.at[s])
    plsc.subcore_barrier()

    pltpu.sync_copy(sh_blk, blkv)
    col = jnp.zeros((16,), jnp.int32)
    pre = jnp.zeros((16,), jnp.int32)
    for r in range(NSUB):
        row = blkv[r, :]
        col = col + row
        pre = pre + jnp.where(r < s, row, 0)
    base_e = plsc.cumsum(col) - col
    rg = base_e + pre
    rl = plsc.cumsum(bsum) - bsum
    for lane in range(16):
        g2d[lane, :] = rg
        d2d[lane, :] = rg - rl
        hrow = h2d[lane, :]
        rg = rg + hrow
        rl = rl + hrow

    @pl.when((c == 0) & (s == 0))
    def _():
        cnt_v[...] = col.astype(jnp.float32)
        pltpu.sync_copy(cnt_v, cnt_out)

    def body(t, cc):
        for u in range(2):
            idxs = seg + (t * 2 + u)
            v = plsc.load_gather(ev, [idxs >> 7, idxs & 127])
            gdst = plsc.load_gather(g2d, [iota, v])
            dl = plsc.load_gather(d2d, [iota, v])
            ldst = gdst - dl
            sc = plsc.load_gather(sv, [idxs >> 7, idxs & 127])
            i0 = ldst >> 7
            i1 = ldst & 127
            plsc.store_scatter(loc_idx, [i0, i1], base_elem + idxs)
            plsc.store_scatter(loc_dst, [i0, i1], gdst)
            plsc.store_scatter(loc_sc, [i0, i1], sc)
            plsc.addupdate_scatter(g2d, [iota, v], ones)
        return cc

    lax.fori_loop(0, SEG // 2, body, 0)

    scats = []
    for j in range(ROWS):
        scats.append(pltpu.async_copy(loc_idx.at[j], sh_idx.at[loc_dst.at[j]], sem))
        scats.append(pltpu.async_copy(loc_sc.at[j], sh_sc.at[loc_dst.at[j]], sem))
    for cp in scats:
        cp.wait()
    plsc.subcore_barrier()

    @pl.when(c == 0)
    def _():
        pltpu.sync_copy(sh_idx.at[pl.ds(base_elem, CH)], bi)
        pltpu.sync_copy(bi, idx_out.at[pl.ds(base_elem, CH)])

    @pl.when(c == 1)
    def _():
        pltpu.sync_copy(sh_sc.at[pl.ds(base_elem, CH)], bs)
        pltpu.sync_copy(bs, sc_out.at[pl.ds(base_elem, CH)])


@jax.jit
def _token_reorder(top_scores, selected_experts_indices):
    eids = selected_experts_indices.reshape(N // 128, 128)
    scores = top_scores.reshape(N // 128, 128)
    return _reorder_kernel(eids, scores)


def kernel(top_scores, selected_experts_indices):
    return _token_reorder(top_scores, selected_experts_indices)
